# baseline (speedup 1.0000x reference)
import jax
import jax.numpy as jnp
from jax import lax
from jax.experimental import pallas as pl
from jax.experimental.pallas import tpu as pltpu
from jax.experimental.pallas import tpu_sc as plsc

_NC = 2
_NS = 16
_NW = _NC * _NS
_G = 2
_BT = 8
_ST = 192


def _tc_body(x_ref, emb_ref, o_ref):
    o_ref[...] = x_ref[...] + emb_ref[...]


def _tc_add(x, emb_weight):
    nt, nf, d = x.shape
    return pl.pallas_call(
        _tc_body,
        grid=(nt // _BT,),
        in_specs=[
            pl.BlockSpec((_BT, nf, d), lambda i: (i, 0, 0)),
            pl.BlockSpec((nf, d), lambda i: (0, 0)),
        ],
        out_specs=pl.BlockSpec((_BT, nf, d), lambda i: (i, 0, 0)),
        out_shape=jax.ShapeDtypeStruct((nt, nf, d), x.dtype),
    )(x, emb_weight)


def _sc_body(fb, d, nch, x_ref, emb_ref, o_ref, emb_v, in_buf, out_buf,
             in_sem0, in_sem1, out_sem0, out_sem1):
    c = lax.axis_index("c")
    s = lax.axis_index("s")
    f0 = (s * _NC + c) * fb
    in_sems = (in_sem0, in_sem1)
    out_sems = (out_sem0, out_sem1)
    nd16 = d // 16

    pltpu.sync_copy(emb_ref.at[pl.ds(f0, fb)], emb_v)

    def in_copy(i, b):
        return pltpu.make_async_copy(
            x_ref.at[pl.ds(i * _G, _G), pl.ds(f0, fb)],
            in_buf.at[b], in_sems[b])

    def out_copy(i, b):
        return pltpu.make_async_copy(
            out_buf.at[b],
            o_ref.at[pl.ds(i * _G, _G), pl.ds(f0, fb)],
            out_sems[b])

    in_copy(0, 0).start()
    in_copy(1, 1).start()

    def step(i, b):
        in_copy(i, b).wait()

        @pl.when(i >= 2)
        def _():
            out_copy(i - 2, b).wait()

        def fbody(f, carry):
            for l in range(nd16):
                ds = pl.ds(l * 16, 16)
                e = emb_v[f, ds]
                for g in range(_G):
                    out_buf[b, g, f, ds] = in_buf[b, g, f, ds] + e
            return carry

        lax.fori_loop(0, fb, fbody, 0)

        out_copy(i, b).start()

        @pl.when(i + 2 < nch)
        def _():
            in_copy(i + 2, b).start()

    def kbody(k, carry):
        step(k * 2, 0)
        step(k * 2 + 1, 1)
        return carry

    lax.fori_loop(0, nch // 2, kbody, 0)

    out_copy(nch - 2, 0).wait()
    out_copy(nch - 1, 1).wait()


def _sc_add(x, emb_weight):
    nt, nf, d = x.shape
    fb = nf // _NW
    nch = nt // _G
    assert nf % _NW == 0 and nt % (2 * _G) == 0 and d % 16 == 0

    body = lambda *refs: _sc_body(fb, d, nch, *refs)
    return pl.kernel(
        body,
        out_type=jax.ShapeDtypeStruct((nt, nf, d), x.dtype),
        mesh=plsc.VectorSubcoreMesh(core_axis_name="c", subcore_axis_name="s"),
        scratch_types=[
            pltpu.VMEM((fb, d), jnp.float32),
            pltpu.VMEM((2, _G, fb, d), jnp.float32),
            pltpu.VMEM((2, _G, fb, d), jnp.float32),
            pltpu.SemaphoreType.DMA,
            pltpu.SemaphoreType.DMA,
            pltpu.SemaphoreType.DMA,
            pltpu.SemaphoreType.DMA,
        ],
    )(x, emb_weight)


def kernel(x, freqs, emb_weight):
    del freqs
    tc_out = _tc_add(x[:_ST], emb_weight)
    sc_out = _sc_add(x[_ST:], emb_weight)
    return jnp.concatenate([tc_out, sc_out], axis=0)


# speedup vs baseline: 8.1928x; 8.1928x over previous
import jax
import jax.numpy as jnp
from jax.experimental import pallas as pl


_BT = 8


def _tc_body(x_ref, emb_ref, o_ref):
    o_ref[...] = x_ref[...] + emb_ref[...]


def kernel(x, freqs, emb_weight):
    del freqs
    nt, nf, d = x.shape
    xt = jnp.transpose(x, (0, 2, 1))
    embt = jnp.transpose(emb_weight, (1, 0))
    outt = pl.pallas_call(
        _tc_body,
        grid=(nt // _BT,),
        in_specs=[
            pl.BlockSpec((_BT, d, nf), lambda i: (i, 0, 0)),
            pl.BlockSpec((d, nf), lambda i: (0, 0)),
        ],
        out_specs=pl.BlockSpec((_BT, d, nf), lambda i: (i, 0, 0)),
        out_shape=jax.ShapeDtypeStruct((nt, d, nf), x.dtype),
    )(xt, embt)
    return jnp.transpose(outt, (0, 2, 1))
